# baseline (speedup 1.0000x reference)
import functools

import jax
import jax.numpy as jnp
from jax import lax
from jax.experimental import pallas as pl
from jax.experimental.pallas import tpu as pltpu
from jax.experimental.pallas import tpu_sc as plsc

D = 64
NEG = 10
NC = 2
NS = 16
NW = NC * NS
CH = 128


def _sc_gather_body(nodes_u, nodes_v, pot_idx, delta, protos, u_emb, v_emb,
                    u_review, v_review,
                    out_protos, out_uid, out_vmix, out_urev, out_vrev, out_pmean,
                    idx_u, idx_v, pidx, dbuf, mbuf, pbuf, psub, vmsub, sem):
    wid = lax.axis_index("s") * NC + lax.axis_index("c")
    n_ch = idx_u.shape[0]
    bpw = n_ch * CH
    base = wid * bpw
    cbase = wid * n_ch

    pltpu.sync_copy(nodes_u.at[pl.ds(cbase, n_ch)], idx_u)
    pltpu.sync_copy(nodes_v.at[pl.ds(cbase, n_ch)], idx_v)
    pltpu.sync_copy(pot_idx.at[pl.ds(cbase * NEG, n_ch * NEG)], pidx)
    pltpu.sync_copy(delta.at[pl.ds(base, bpw)], dbuf)

    for table, idx, out in ((u_emb, idx_u, out_uid),
                            (u_review, idx_u, out_urev),
                            (protos, idx_u, out_protos),
                            (v_review, idx_v, out_vrev)):
        for c in range(n_ch):
            pltpu.async_copy(table.at[idx.at[c]], mbuf, sem).wait()
            pltpu.sync_copy(mbuf, out.at[pl.ds(base + c * CH, CH)])

    for c in range(n_ch):
        cps = []
        for g in range(NEG):
            cps.append(pltpu.async_copy(
                v_emb.at[pidx.at[c * NEG + g]],
                pbuf.at[pl.ds(g * CH, CH)], sem))
        cps.append(pltpu.async_copy(v_emb.at[idx_v.at[c]], mbuf, sem))
        for cp in cps:
            cp.wait()

        def body(r, carry):
            dl = dbuf[c * CH + r, :]
            one_m = jnp.float32(1.0) - dl
            for grp in range(D // 16):
                sl = pl.ds(grp * 16, 16)
                acc = pbuf[r * NEG, sl]
                for j in range(1, NEG):
                    acc = acc + pbuf[r * NEG + j, sl]
                m = acc / jnp.float32(NEG)
                psub[r, sl] = m
                vmsub[r, sl] = dl * mbuf[r, sl] + one_m * m
            return carry

        lax.fori_loop(0, CH, body, jnp.int32(0))

        pltpu.sync_copy(psub, out_pmean.at[pl.ds(base + c * CH, CH)])
        pltpu.sync_copy(vmsub, out_vmix.at[pl.ds(base + c * CH, CH)])


def _make_sc_gather(B):
    n_ch = B // (NW * CH)
    out = jax.ShapeDtypeStruct((B, D), jnp.float32)
    return pl.kernel(
        _sc_gather_body,
        out_type=(out,) * 6,
        mesh=plsc.VectorSubcoreMesh(core_axis_name="c", subcore_axis_name="s"),
        scratch_types=[
            pltpu.VMEM((n_ch, CH), jnp.int32),
            pltpu.VMEM((n_ch, CH), jnp.int32),
            pltpu.VMEM((n_ch * NEG, CH), jnp.int32),
            pltpu.VMEM((n_ch * CH, 16), jnp.float32),
            pltpu.VMEM((CH, D), jnp.float32),
            pltpu.VMEM((CH * NEG, D), jnp.float32),
            pltpu.VMEM((CH, D), jnp.float32),
            pltpu.VMEM((CH, D), jnp.float32),
            pltpu.SemaphoreType.DMA,
        ],
        compiler_params=pltpu.CompilerParams(use_tc_tiling_on_sc=False),
    )


def _mlp_body(uid, vmix, w1u, w1v, b1, g1, be1, w2, b2, g2, be2,
              w3, b3, g3, be3, wc, bc, out):
    hi = lax.Precision.HIGHEST

    def bn(x, gamma, beta):
        m = jnp.mean(x, axis=0, keepdims=True)
        v = jnp.mean((x - m) * (x - m), axis=0, keepdims=True)
        return (x - m) / jnp.sqrt(v + jnp.float32(1e-5)) * gamma + beta

    x1 = (jnp.dot(uid[...], w1u[...], precision=hi)
          + jnp.dot(vmix[...], w1v[...], precision=hi))
    h1 = bn(jnp.maximum(x1 + b1[...], 0.0), g1[...], be1[...])
    h2 = bn(jnp.maximum(jnp.dot(h1, w2[...], precision=hi) + b2[...], 0.0),
            g2[...], be2[...])
    h3 = bn(jnp.maximum(jnp.dot(h2, w3[...], precision=hi) + b3[...], 0.0),
            g3[...], be3[...])
    logit = jnp.dot(h3, wc[...], precision=hi) + bc[...]
    out[...] = jax.nn.sigmoid(logit)


def kernel(nodes_u, nodes_v, potential_items, inter_nums, delta, global_protos,
           u_emb, v_emb, u_review, v_review,
           W1, b1, g1, be1, W2, b2, g2, be2, W3, b3, g3, be3, Wc, bc):
    B = nodes_u.shape[0]
    nodes_u2 = nodes_u.reshape(B // CH, CH)
    nodes_v2 = nodes_v.reshape(B // CH, CH)
    pot2 = potential_items.reshape(B * NEG // CH, CH)
    delta16 = jnp.broadcast_to(delta.reshape(B, 1), (B, 16))

    u_feats, u_id_feats, v_mixed, u_review_feats, v_review_feats, pmean = (
        _make_sc_gather(B)(nodes_u2, nodes_v2, pot2, delta16, global_protos,
                           u_emb, v_emb, u_review, v_review))

    pred = pl.pallas_call(
        _mlp_body,
        out_shape=jax.ShapeDtypeStruct((B, 1), jnp.float32),
        compiler_params=pltpu.CompilerParams(
            vmem_limit_bytes=100 * 1024 * 1024),
    )(u_id_feats, v_mixed, W1[:D], W1[D:], b1.reshape(1, D), g1.reshape(1, D),
      be1.reshape(1, D), W2, b2.reshape(1, D // 2), g2.reshape(1, D // 2),
      be2.reshape(1, D // 2), W3, b3.reshape(1, D // 4), g3.reshape(1, D // 4),
      be3.reshape(1, D // 4), Wc, bc.reshape(1, 1))

    return (u_feats, pred.reshape(B), u_id_feats, v_mixed,
            u_review_feats, v_review_feats, pmean)
